# baseline (speedup 1.0000x reference)
import functools

import jax
import jax.numpy as jnp
from jax import lax
from jax.experimental import pallas as pl
from jax.experimental.pallas import tpu as pltpu
from jax.experimental.pallas import tpu_sc as plsc

N_NODES = 10000
N_EDGES = 160000
DIM = 256
HALF = 128
N_TILES = 16
OPW = 64
EPAD = 163840
EROWS = EPAD // OPW
ROWS_PER_TILE = EROWS // N_TILES
PAIR_ROWS = 40
N_PAIRS = ROWS_PER_TILE // PAIR_ROWS
NSLOT = 4
GLEAD = 2
DLAG = 2
PAD_ROWS = 48
ACC_ROWS = N_NODES + PAD_ROWS
WB_TILES = 10
WB_ROWS = N_NODES // WB_TILES
ZDEG = 1008


def _make_sc_agg(want_deg: bool):
    out_type = [
        jax.ShapeDtypeStruct((N_NODES, HALF), jnp.float32),
        jax.ShapeDtypeStruct((N_NODES, HALF), jnp.float32),
    ]
    if want_deg:
        out_type.append(jax.ShapeDtypeStruct((N_NODES,), jnp.float32))
    scratch = (
        [pltpu.VMEM((PAIR_ROWS, OPW), jnp.int32) for _ in range(2)]
        + [pltpu.VMEM((OPW, HALF), jnp.float32) for _ in range(NSLOT)]
        + [pltpu.VMEM_SHARED((ACC_ROWS, HALF), jnp.float32)]
        + [pltpu.SemaphoreType.DMA for _ in range(2 * NSLOT)]
    )
    if want_deg:
        scratch += [
            pltpu.VMEM((OPW,), jnp.float32),
            pltpu.VMEM((ZDEG,), jnp.float32),
            pltpu.VMEM_SHARED((ACC_ROWS,), jnp.float32),
        ] + [pltpu.SemaphoreType.DMA for _ in range(NSLOT)]
    mesh = plsc.VectorSubcoreMesh(core_axis_name="c", subcore_axis_name="s")

    @functools.partial(pl.kernel, mesh=mesh, out_type=out_type,
                       scratch_types=scratch)
    def sc_agg(hL, hR, src_hbm, dst_hbm, zeros_hbm, *refs):
        if want_deg:
            aggL, aggR, deg_out = refs[:3]
            refs = refs[3:]
        else:
            aggL, aggR = refs[:2]
            refs = refs[2:]
        srcb, dstb = refs[:2]
        rows_l = refs[2:2 + NSLOT]
        acc = refs[2 + NSLOT]
        gsem = refs[3 + NSLOT:3 + 2 * NSLOT]
        ssem = refs[3 + 2 * NSLOT:3 + 3 * NSLOT]
        if want_deg:
            ones_v, zdeg, dacc = refs[3 + 3 * NSLOT:6 + 3 * NSLOT]
            dsem = refs[6 + 3 * NSLOT:6 + 4 * NSLOT]
        c = lax.axis_index("c")
        s = lax.axis_index("s")
        row0 = s * WB_ROWS

        @pl.when(s < WB_TILES)
        def _():
            pltpu.sync_copy(zeros_hbm, acc.at[pl.ds(row0, WB_ROWS)])
        if want_deg:
            @pl.when(jnp.logical_and(c == 0, s < WB_TILES))
            def _():
                def df(i, carry):
                    zdeg[pl.ds(i * 16, 16)] = jnp.zeros((16,), jnp.float32)
                    return carry
                lax.fori_loop(0, ZDEG // 16, df, 0)
                pltpu.sync_copy(zdeg.at[pl.ds(0, WB_ROWS)],
                                dacc.at[pl.ds(row0, WB_ROWS)])

            @pl.when(c == 0)
            def _():
                def of(i, carry):
                    ones_v[pl.ds(i * 16, 16)] = jnp.ones((16,), jnp.float32)
                    return carry
                lax.fori_loop(0, OPW // 16, of, 0)
        plsc.subcore_barrier()

        rbase = s * ROWS_PER_TILE

        def _issue_gather(t, slot):
            @pl.when(c == 0)
            def _():
                pltpu.async_copy(hL.at[srcb.at[t]], rows_l[slot], gsem[slot])

            @pl.when(c == 1)
            def _():
                pltpu.async_copy(hR.at[srcb.at[t]], rows_l[slot], gsem[slot])

        def _wait_gather(t, slot):
            pltpu.make_async_copy(hL.at[srcb.at[t]], rows_l[slot],
                                  gsem[slot]).wait()

        def _issue_scatter(t, slot):
            pltpu.async_copy(rows_l[slot], acc.at[dstb.at[t]], ssem[slot],
                             add=True)
            if want_deg:
                @pl.when(c == 0)
                def _():
                    pltpu.async_copy(ones_v, dacc.at[dstb.at[t]], dsem[slot],
                                     add=True)

        def _drain_scatter(t, slot):
            pltpu.make_async_copy(rows_l[slot], acc.at[dstb.at[t]],
                                  ssem[slot]).wait()
            if want_deg:
                @pl.when(c == 0)
                def _():
                    pltpu.make_async_copy(ones_v, dacc.at[dstb.at[t]],
                                          dsem[slot]).wait()

        def _drain_tail():
            for t in range(PAIR_ROWS - DLAG, PAIR_ROWS):
                _drain_scatter(t, t % NSLOT)

        def pair_body(p, carry):
            @pl.when(p > 0)
            def _():
                _drain_tail()
            r = rbase + p * PAIR_ROWS
            pltpu.sync_copy(src_hbm.at[pl.ds(r, PAIR_ROWS)], srcb)
            pltpu.sync_copy(dst_hbm.at[pl.ds(r, PAIR_ROWS)], dstb)
            for t in range(GLEAD):
                _issue_gather(t, t % NSLOT)
            for t in range(PAIR_ROWS):
                slot = t % NSLOT
                _wait_gather(t, slot)
                _issue_scatter(t, slot)
                if t >= DLAG:
                    _drain_scatter(t - DLAG, (t - DLAG) % NSLOT)
                nt = t + GLEAD
                if nt < PAIR_ROWS:
                    _issue_gather(nt, nt % NSLOT)
            return carry
        lax.fori_loop(0, N_PAIRS, pair_body, 0)
        _drain_tail()
        plsc.subcore_barrier()

        rows = pl.ds(row0, WB_ROWS)

        @pl.when(jnp.logical_and(c == 0, s < WB_TILES))
        def _():
            pltpu.sync_copy(acc.at[rows], aggL.at[rows])
            if want_deg:
                pltpu.sync_copy(dacc.at[rows], zdeg.at[pl.ds(0, WB_ROWS)])
                pltpu.sync_copy(zdeg.at[pl.ds(0, WB_ROWS)], deg_out.at[rows])

        @pl.when(jnp.logical_and(c == 1, s < WB_TILES))
        def _():
            pltpu.sync_copy(acc.at[rows], aggR.at[rows])

    return sc_agg


def _make_tc_layer(relu: bool, split_out: bool, rows_blk: int = 1000):
    grid = (N_NODES // rows_blk,)

    def body(hL, hR, aL, aR, dg, wsl, wsr, wnl, wnr, b, *o):
        inv = 1.0 / jnp.maximum(dg[...], 1.0)
        acc = jnp.dot(hL[...], wsl[...], preferred_element_type=jnp.float32)
        acc = acc + jnp.dot(hR[...], wsr[...], preferred_element_type=jnp.float32)
        acc = acc + jnp.dot(aL[...] * inv, wnl[...], preferred_element_type=jnp.float32)
        acc = acc + jnp.dot(aR[...] * inv, wnr[...], preferred_element_type=jnp.float32)
        acc = acc + b[...]
        if relu:
            acc = jnp.maximum(acc, 0.0)
        if split_out:
            o[0][...] = acc[:, :HALF]
            o[1][...] = acc[:, HALF:]
        else:
            o[0][...] = acc

    half_spec = pl.BlockSpec((rows_blk, HALF), lambda i: (i, 0))
    in_specs = [
        half_spec, half_spec, half_spec, half_spec,
        pl.BlockSpec((rows_blk, 1), lambda i: (i, 0)),
        pl.BlockSpec((HALF, DIM), lambda i: (0, 0)),
        pl.BlockSpec((HALF, DIM), lambda i: (0, 0)),
        pl.BlockSpec((HALF, DIM), lambda i: (0, 0)),
        pl.BlockSpec((HALF, DIM), lambda i: (0, 0)),
        pl.BlockSpec((1, DIM), lambda i: (0, 0)),
    ]
    if split_out:
        out_shape = [jax.ShapeDtypeStruct((N_NODES, HALF), jnp.float32)] * 2
        out_specs = [half_spec, half_spec]
    else:
        out_shape = [jax.ShapeDtypeStruct((N_NODES, DIM), jnp.float32)]
        out_specs = [pl.BlockSpec((rows_blk, DIM), lambda i: (i, 0))]
    return pl.pallas_call(body, grid=grid, in_specs=in_specs,
                          out_specs=out_specs, out_shape=out_shape)


_sc_agg_deg = _make_sc_agg(want_deg=True)
_sc_agg = _make_sc_agg(want_deg=False)
_tc_hidden = _make_tc_layer(relu=True, split_out=True)
_tc_final = _make_tc_layer(relu=False, split_out=False)


def _split(w):
    return w[:HALF], w[HALF:]


def kernel(x, edge_index, W_self0, W_neigh0, b0, W_self1, W_neigh1, b1,
           W_self2, W_neigh2, b2):
    src = edge_index[0].astype(jnp.int32)
    dst = edge_index[1].astype(jnp.int32)
    npad = EPAD - N_EDGES
    src2d = jnp.concatenate(
        [src, jnp.zeros((npad,), jnp.int32)]).reshape(EROWS, OPW)
    dst2d = jnp.concatenate(
        [dst, N_NODES + (jnp.arange(npad, dtype=jnp.int32) % PAD_ROWS)]
    ).reshape(EROWS, OPW)
    zeros = jnp.zeros((WB_ROWS, HALF), jnp.float32)
    xL, xR = x[:, :HALF], x[:, HALF:]

    a1L, a1R, deg1d = _sc_agg_deg(xL, xR, src2d, dst2d, zeros)
    deg = deg1d.reshape(N_NODES, 1)
    h1L, h1R = _tc_hidden(xL, xR, a1L, a1R, deg,
                          *_split(W_self0), *_split(W_neigh0),
                          b0.reshape(1, DIM))
    a2L, a2R = _sc_agg(h1L, h1R, src2d, dst2d, zeros)
    h2L, h2R = _tc_hidden(h1L, h1R, a2L, a2R, deg,
                          *_split(W_self1), *_split(W_neigh1),
                          b1.reshape(1, DIM))
    a3L, a3R = _sc_agg(h2L, h2R, src2d, dst2d, zeros)
    (out,) = _tc_final(h2L, h2R, a3L, a3R, deg,
                       *_split(W_self2), *_split(W_neigh2),
                       b2.reshape(1, DIM))
    return out


# speedup vs baseline: 1.0057x; 1.0057x over previous
import functools

import jax
import jax.numpy as jnp
from jax import lax
from jax.experimental import pallas as pl
from jax.experimental.pallas import tpu as pltpu
from jax.experimental.pallas import tpu_sc as plsc

N_NODES = 10000
N_EDGES = 160000
DIM = 256
HALF = 128
N_TILES = 16
OPW = 40
EPAD = 163840
EROWS = EPAD // OPW
ROWS_PER_TILE = EROWS // N_TILES
PAIR_ROWS = 32
N_PAIRS = ROWS_PER_TILE // PAIR_ROWS
NSLOT = 8
GLEAD = 4
DLAG = 4
PAD_ROWS = 16
ACC_ROWS = N_NODES + PAD_ROWS
WB_TILES = 10
WB_ROWS = N_NODES // WB_TILES
ZDEG = 1008


def _make_sc_agg(want_deg: bool):
    out_type = [
        jax.ShapeDtypeStruct((N_NODES, HALF), jnp.float32),
        jax.ShapeDtypeStruct((N_NODES, HALF), jnp.float32),
    ]
    if want_deg:
        out_type.append(jax.ShapeDtypeStruct((N_NODES,), jnp.float32))
    scratch = (
        [pltpu.VMEM((PAIR_ROWS, OPW), jnp.int32) for _ in range(2)]
        + [pltpu.VMEM((OPW, HALF), jnp.float32) for _ in range(NSLOT)]
        + [pltpu.VMEM_SHARED((ACC_ROWS, HALF), jnp.float32)]
        + [pltpu.SemaphoreType.DMA for _ in range(2 * NSLOT)]
    )
    if want_deg:
        scratch += [
            pltpu.VMEM((48,), jnp.float32),
            pltpu.VMEM((ZDEG,), jnp.float32),
            pltpu.VMEM_SHARED((ACC_ROWS,), jnp.float32),
        ] + [pltpu.SemaphoreType.DMA for _ in range(NSLOT)]
    mesh = plsc.VectorSubcoreMesh(core_axis_name="c", subcore_axis_name="s")

    @functools.partial(pl.kernel, mesh=mesh, out_type=out_type,
                       scratch_types=scratch)
    def sc_agg(hL, hR, src_hbm, dst_hbm, zeros_hbm, *refs):
        if want_deg:
            aggL, aggR, deg_out = refs[:3]
            refs = refs[3:]
        else:
            aggL, aggR = refs[:2]
            refs = refs[2:]
        srcb, dstb = refs[:2]
        rows_l = refs[2:2 + NSLOT]
        acc = refs[2 + NSLOT]
        gsem = refs[3 + NSLOT:3 + 2 * NSLOT]
        ssem = refs[3 + 2 * NSLOT:3 + 3 * NSLOT]
        if want_deg:
            ones_v, zdeg, dacc = refs[3 + 3 * NSLOT:6 + 3 * NSLOT]
            dsem = refs[6 + 3 * NSLOT:6 + 4 * NSLOT]
        c = lax.axis_index("c")
        s = lax.axis_index("s")
        row0 = s * WB_ROWS

        @pl.when(s < WB_TILES)
        def _():
            pltpu.sync_copy(zeros_hbm, acc.at[pl.ds(row0, WB_ROWS)])
        if want_deg:
            @pl.when(jnp.logical_and(c == 0, s < WB_TILES))
            def _():
                def df(i, carry):
                    zdeg[pl.ds(i * 16, 16)] = jnp.zeros((16,), jnp.float32)
                    return carry
                lax.fori_loop(0, ZDEG // 16, df, 0)
                pltpu.sync_copy(zdeg.at[pl.ds(0, WB_ROWS)],
                                dacc.at[pl.ds(row0, WB_ROWS)])

            @pl.when(c == 0)
            def _():
                def of(i, carry):
                    ones_v[pl.ds(i * 16, 16)] = jnp.ones((16,), jnp.float32)
                    return carry
                lax.fori_loop(0, 48 // 16, of, 0)
        plsc.subcore_barrier()

        rbase = s * ROWS_PER_TILE

        def _issue_gather(t, slot):
            @pl.when(c == 0)
            def _():
                pltpu.async_copy(hL.at[srcb.at[t]], rows_l[slot], gsem[slot])

            @pl.when(c == 1)
            def _():
                pltpu.async_copy(hR.at[srcb.at[t]], rows_l[slot], gsem[slot])

        def _wait_gather(t, slot):
            pltpu.make_async_copy(hL.at[srcb.at[t]], rows_l[slot],
                                  gsem[slot]).wait()

        def _issue_scatter(t, slot):
            pltpu.async_copy(rows_l[slot], acc.at[dstb.at[t]], ssem[slot],
                             add=True)
            if want_deg:
                @pl.when(c == 0)
                def _():
                    pltpu.async_copy(ones_v.at[pl.ds(0, OPW)],
                                     dacc.at[dstb.at[t]], dsem[slot],
                                     add=True)

        def _drain_scatter(t, slot):
            pltpu.make_async_copy(rows_l[slot], acc.at[dstb.at[t]],
                                  ssem[slot]).wait()
            if want_deg:
                @pl.when(c == 0)
                def _():
                    pltpu.make_async_copy(ones_v.at[pl.ds(0, OPW)],
                                          dacc.at[dstb.at[t]],
                                          dsem[slot]).wait()

        def _drain_tail():
            for t in range(PAIR_ROWS - DLAG, PAIR_ROWS):
                _drain_scatter(t, t % NSLOT)

        def pair_body(p, carry):
            @pl.when(p > 0)
            def _():
                _drain_tail()
            r = rbase + p * PAIR_ROWS
            pltpu.sync_copy(src_hbm.at[pl.ds(r, PAIR_ROWS)], srcb)
            pltpu.sync_copy(dst_hbm.at[pl.ds(r, PAIR_ROWS)], dstb)
            for t in range(GLEAD):
                _issue_gather(t, t % NSLOT)
            for t in range(PAIR_ROWS):
                slot = t % NSLOT
                _wait_gather(t, slot)
                _issue_scatter(t, slot)
                if t >= DLAG:
                    _drain_scatter(t - DLAG, (t - DLAG) % NSLOT)
                nt = t + GLEAD
                if nt < PAIR_ROWS:
                    _issue_gather(nt, nt % NSLOT)
            return carry
        lax.fori_loop(0, N_PAIRS, pair_body, 0)
        _drain_tail()
        plsc.subcore_barrier()

        rows = pl.ds(row0, WB_ROWS)

        @pl.when(jnp.logical_and(c == 0, s < WB_TILES))
        def _():
            pltpu.sync_copy(acc.at[rows], aggL.at[rows])
            if want_deg:
                pltpu.sync_copy(dacc.at[rows], zdeg.at[pl.ds(0, WB_ROWS)])
                pltpu.sync_copy(zdeg.at[pl.ds(0, WB_ROWS)], deg_out.at[rows])

        @pl.when(jnp.logical_and(c == 1, s < WB_TILES))
        def _():
            pltpu.sync_copy(acc.at[rows], aggR.at[rows])

    return sc_agg


def _make_tc_layer(relu: bool, split_out: bool, rows_blk: int = 1000):
    grid = (N_NODES // rows_blk,)

    def body(hL, hR, aL, aR, dg, wsl, wsr, wnl, wnr, b, *o):
        inv = 1.0 / jnp.maximum(dg[...], 1.0)
        acc = jnp.dot(hL[...], wsl[...], preferred_element_type=jnp.float32)
        acc = acc + jnp.dot(hR[...], wsr[...], preferred_element_type=jnp.float32)
        acc = acc + jnp.dot(aL[...] * inv, wnl[...], preferred_element_type=jnp.float32)
        acc = acc + jnp.dot(aR[...] * inv, wnr[...], preferred_element_type=jnp.float32)
        acc = acc + b[...]
        if relu:
            acc = jnp.maximum(acc, 0.0)
        if split_out:
            o[0][...] = acc[:, :HALF]
            o[1][...] = acc[:, HALF:]
        else:
            o[0][...] = acc

    half_spec = pl.BlockSpec((rows_blk, HALF), lambda i: (i, 0))
    in_specs = [
        half_spec, half_spec, half_spec, half_spec,
        pl.BlockSpec((rows_blk, 1), lambda i: (i, 0)),
        pl.BlockSpec((HALF, DIM), lambda i: (0, 0)),
        pl.BlockSpec((HALF, DIM), lambda i: (0, 0)),
        pl.BlockSpec((HALF, DIM), lambda i: (0, 0)),
        pl.BlockSpec((HALF, DIM), lambda i: (0, 0)),
        pl.BlockSpec((1, DIM), lambda i: (0, 0)),
    ]
    if split_out:
        out_shape = [jax.ShapeDtypeStruct((N_NODES, HALF), jnp.float32)] * 2
        out_specs = [half_spec, half_spec]
    else:
        out_shape = [jax.ShapeDtypeStruct((N_NODES, DIM), jnp.float32)]
        out_specs = [pl.BlockSpec((rows_blk, DIM), lambda i: (i, 0))]
    return pl.pallas_call(body, grid=grid, in_specs=in_specs,
                          out_specs=out_specs, out_shape=out_shape)


_sc_agg_deg = _make_sc_agg(want_deg=True)
_sc_agg = _make_sc_agg(want_deg=False)
_tc_hidden = _make_tc_layer(relu=True, split_out=True)
_tc_final = _make_tc_layer(relu=False, split_out=False)


def _split(w):
    return w[:HALF], w[HALF:]


def kernel(x, edge_index, W_self0, W_neigh0, b0, W_self1, W_neigh1, b1,
           W_self2, W_neigh2, b2):
    src = edge_index[0].astype(jnp.int32)
    dst = edge_index[1].astype(jnp.int32)
    npad = EPAD - N_EDGES
    src2d = jnp.concatenate(
        [src, jnp.zeros((npad,), jnp.int32)]).reshape(EROWS, OPW)
    dst2d = jnp.concatenate(
        [dst, N_NODES + (jnp.arange(npad, dtype=jnp.int32) % PAD_ROWS)]
    ).reshape(EROWS, OPW)
    zeros = jnp.zeros((WB_ROWS, HALF), jnp.float32)
    xL, xR = x[:, :HALF], x[:, HALF:]

    a1L, a1R, deg1d = _sc_agg_deg(xL, xR, src2d, dst2d, zeros)
    deg = deg1d.reshape(N_NODES, 1)
    h1L, h1R = _tc_hidden(xL, xR, a1L, a1R, deg,
                          *_split(W_self0), *_split(W_neigh0),
                          b0.reshape(1, DIM))
    a2L, a2R = _sc_agg(h1L, h1R, src2d, dst2d, zeros)
    h2L, h2R = _tc_hidden(h1L, h1R, a2L, a2R, deg,
                          *_split(W_self1), *_split(W_neigh1),
                          b1.reshape(1, DIM))
    a3L, a3R = _sc_agg(h2L, h2R, src2d, dst2d, zeros)
    (out,) = _tc_final(h2L, h2R, a3L, a3R, deg,
                       *_split(W_self2), *_split(W_neigh2),
                       b2.reshape(1, DIM))
    return out
